# baseline (speedup 1.0000x reference)
import functools

import jax
import jax.numpy as jnp
from jax import lax
from jax.experimental import pallas as pl
from jax.experimental.pallas import tpu as pltpu
from jax.experimental.pallas import tpu_sc as plsc

_D = 32
_T = 200
_NC = 2
_NS = 16
_NW = _NC * _NS
_NT = 4
_DT = _D // 8
_NBUF = 2


def _gather_body(idx_hbm, table_hbm, out_hbm, *scratch):
    idx_v, rows_v, slab_v = scratch[0], scratch[1], scratch[2]
    sem_idx = scratch[3:3 + _NBUF]
    sem_gat = scratch[3 + _NBUF:3 + 2 * _NBUF]
    sem_out = scratch[3 + 2 * _NBUF:3 + 3 * _NBUF]

    wid = lax.axis_index("s") * _NC + lax.axis_index("c")
    it0 = wid * _NT

    def start_idx(c, b):
        pltpu.async_copy(
            idx_hbm.at[c, pl.ds(it0, _NT)], idx_v.at[b], sem_idx[b])

    def wait_idx(b):
        pltpu.make_async_copy(
            idx_hbm.at[0, pl.ds(0, _NT)], idx_v.at[b], sem_idx[b]).wait()

    def fire_gathers(b):
        for itl in range(_NT):
            pltpu.async_copy(
                table_hbm.at[idx_v.at[b, itl]], rows_v.at[b, itl], sem_gat[b])

    def wait_gat(b):
        for itl in range(_NT):
            pltpu.make_async_copy(
                table_hbm.at[pl.ds(0, 128)], rows_v.at[b, itl],
                sem_gat[b]).wait()

    def start_out(c, b):
        for dt in range(_DT):
            pltpu.async_copy(
                slab_v.at[b, dt], out_hbm.at[c, dt, pl.ds(it0, _NT)],
                sem_out[b])

    def wait_out(b):
        for dt in range(_DT):
            pltpu.make_async_copy(
                slab_v.at[b, dt], out_hbm.at[0, dt, pl.ds(0, _NT)],
                sem_out[b]).wait()

    def transpose(b):
        iota = lax.iota(jnp.int32, 16)
        b_v = jnp.full((16,), b, jnp.int32)

        def body(ilc, carry):
            il_v = iota + ilc * 16
            for dt in range(_DT):
                for itl in range(_NT):
                    itl_v = jnp.full((16,), itl, jnp.int32)
                    for ds in range(8):
                        d_v = jnp.full((16,), dt * 8 + ds, jnp.int32)
                        v = plsc.load_gather(rows_v, [b_v, itl_v, il_v, d_v])
                        slab_v[b, dt, itl, ds, pl.ds(ilc * 16, 16)] = v
            return carry

        lax.fori_loop(0, 8, body, 0)

    for b in range(_NBUF):
        start_idx(b, b)
    wait_idx(0)
    fire_gathers(0)

    def outer(g, carry):
        for b in range(_NBUF):
            c = g * _NBUF + b
            b1 = (b + 1) % _NBUF
            wait_gat(b)

            @pl.when(c + 1 < _T)
            def _():
                wait_idx(b1)
                fire_gathers(b1)
            @pl.when(c >= _NBUF)
            def _():
                wait_out(b)

            transpose(b)
            start_out(c, b)

            @pl.when(c + _NBUF < _T)
            def _():
                start_idx(c + _NBUF, b)
        return carry

    lax.fori_loop(0, _T // _NBUF, outer, 0)

    for b in range(_NBUF):
        wait_out(b)


@jax.jit
def _run(idx3d, table):
    mesh = plsc.VectorSubcoreMesh(core_axis_name="c", subcore_axis_name="s")
    scratch = [
        pltpu.VMEM((_NBUF, _NT, 128), jnp.int32),
        pltpu.VMEM((_NBUF, _NT, 128, _D), jnp.float32),
        pltpu.VMEM((_NBUF, _DT, _NT, 8, 128), jnp.float32),
    ] + [pltpu.SemaphoreType.DMA] * (3 * _NBUF)
    p = pl.kernel(
        _gather_body,
        mesh=mesh,
        out_type=jax.ShapeDtypeStruct((_T, _DT, 128, 8, 128), jnp.float32),
        scratch_types=scratch,
        compiler_params=pltpu.CompilerParams(
            use_tc_tiling_on_sc=False, needs_layout_passes=False),
    )(idx3d, table)
    return p


def kernel(time_steps, table):
    n_i, n_t = time_steps.shape
    idx3d = time_steps.T.reshape(n_t, n_i // 128, 128).astype(jnp.int32)
    p = _run(idx3d, table)
    return p.transpose(2, 4, 0, 1, 3).reshape(n_i, n_t, _D)


# speedup vs baseline: 6.1570x; 6.1570x over previous
import functools

import jax
import jax.numpy as jnp
from jax import lax
from jax.experimental import pallas as pl
from jax.experimental.pallas import tpu as pltpu
from jax.experimental.pallas import tpu_sc as plsc

_D = 32
_T = 200
_NC = 2
_NS = 16
_NW = _NC * _NS
_NT = 4
_DT = _D // 8
_NBUF = 2
_SLAB = _DT * _NT * 8 * 128


def _gather_body(idx_hbm, table_hbm, out_hbm, *scratch):
    idx_v, rows_v, slab_v = scratch[0], scratch[1], scratch[2]
    sem_idx = scratch[3:3 + _NBUF]
    sem_gat = scratch[3 + _NBUF:3 + 2 * _NBUF]
    sem_out = scratch[3 + 2 * _NBUF:3 + 3 * _NBUF]

    wid = lax.axis_index("s") * _NC + lax.axis_index("c")
    it0 = wid * _NT

    def start_idx(c, b):
        pltpu.async_copy(
            idx_hbm.at[c, pl.ds(it0, _NT)], idx_v.at[b], sem_idx[b])

    def wait_idx(b):
        pltpu.make_async_copy(
            idx_hbm.at[0, pl.ds(0, _NT)], idx_v.at[b], sem_idx[b]).wait()

    def fire_gathers(b):
        for itl in range(_NT):
            pltpu.async_copy(
                table_hbm.at[idx_v.at[b, itl]], rows_v.at[b, itl], sem_gat[b])

    def wait_gat(b):
        for itl in range(_NT):
            pltpu.make_async_copy(
                table_hbm.at[pl.ds(0, 128)], rows_v.at[b, itl],
                sem_gat[b]).wait()

    def start_out(c, b):
        for dt in range(_DT):
            pltpu.async_copy(
                slab_v.at[pl.ds(b * _SLAB + dt * 4096, 4096)],
                out_hbm.at[c, dt, pl.ds(it0 * 1024, 4096)],
                sem_out[b])

    def wait_out(b):
        for dt in range(_DT):
            pltpu.make_async_copy(
                slab_v.at[pl.ds(b * _SLAB + dt * 4096, 4096)],
                out_hbm.at[0, dt, pl.ds(0, 4096)],
                sem_out[b]).wait()

    def transpose(b):
        iota = lax.iota(jnp.int32, 16)
        pat0 = (iota // 8) * 4096 + (iota % 8) * 128 + b * _SLAB
        for itl in range(_NT):
            base = pat0 + itl * 1024

            @functools.partial(plsc.parallel_loop, 0, 128, unroll=8)
            def _(il):
                v0 = rows_v[b, itl, il, pl.ds(0, 16)]
                v1 = rows_v[b, itl, il, pl.ds(16, 16)]
                idx0 = base + il
                plsc.store_scatter(slab_v, [idx0], v0)
                plsc.store_scatter(slab_v, [idx0 + 2 * 4096], v1)

    for b in range(_NBUF):
        start_idx(b, b)
    wait_idx(0)
    fire_gathers(0)

    def outer(g, carry):
        for b in range(_NBUF):
            c = g * _NBUF + b
            b1 = (b + 1) % _NBUF
            wait_gat(b)

            @pl.when(c + 1 < _T)
            def _():
                wait_idx(b1)
                fire_gathers(b1)
            @pl.when(c >= _NBUF)
            def _():
                wait_out(b)

            transpose(b)
            start_out(c, b)

            @pl.when(c + _NBUF < _T)
            def _():
                start_idx(c + _NBUF, b)
        return carry

    lax.fori_loop(0, _T // _NBUF, outer, 0)

    for b in range(_NBUF):
        wait_out(b)


@jax.jit
def _run(idx3d, table):
    mesh = plsc.VectorSubcoreMesh(core_axis_name="c", subcore_axis_name="s")
    scratch = [
        pltpu.VMEM((_NBUF, _NT, 128), jnp.int32),
        pltpu.VMEM((_NBUF, _NT, 128, _D), jnp.float32),
        pltpu.VMEM((_NBUF * _SLAB,), jnp.float32),
    ] + [pltpu.SemaphoreType.DMA] * (3 * _NBUF)
    p = pl.kernel(
        _gather_body,
        mesh=mesh,
        out_type=jax.ShapeDtypeStruct((_T, _DT, 128 * 8 * 128), jnp.float32),
        scratch_types=scratch,
        compiler_params=pltpu.CompilerParams(
            use_tc_tiling_on_sc=False, needs_layout_passes=False),
    )(idx3d, table)
    return p


def kernel(time_steps, table):
    n_i, n_t = time_steps.shape
    idx3d = time_steps.T.reshape(n_t, n_i // 128, 128).astype(jnp.int32)
    p = _run(idx3d, table)
    p5 = p.reshape(n_t, _DT, n_i // 128, 8, 128)
    return p5.transpose(2, 4, 0, 1, 3).reshape(n_i, n_t, _D)
